# Initial kernel scaffold; baseline (speedup 1.0000x reference)
import jax
import jax.numpy as jnp
from jax.experimental import pallas as pl


def kernel(x, embd):
    raise NotImplementedError("write your pallas kernel here")


# speedup vs baseline: 1.3366x; 1.3366x over previous
import functools

import jax
import jax.numpy as jnp
from jax import lax
from jax.experimental import pallas as pl
from jax.experimental.pallas import tpu as pltpu
from jax.experimental.pallas import tpu_sc as plsc


def _emb_lookup(idx, embd, n_rows, d, n_workers, chunk):
    rows_per_w = n_rows // n_workers
    n_chunks = rows_per_w // chunk
    mesh = plsc.VectorSubcoreMesh(core_axis_name="c", subcore_axis_name="s")

    @functools.partial(
        pl.kernel,
        mesh=mesh,
        out_type=jax.ShapeDtypeStruct((n_rows, d), jnp.float32),
        scratch_types=[
            pltpu.VMEM((chunk,), jnp.int32),
            pltpu.VMEM((chunk, d), jnp.float32),
            pltpu.SemaphoreType.DMA,
        ],
        compiler_params=pltpu.CompilerParams(use_tc_tiling_on_sc=False),
    )
    def emb_kernel(idx_hbm, table_hbm, out_hbm, idx_v, rows_v, sem):
        wid = lax.axis_index("s") * 2 + lax.axis_index("c")
        base = wid * rows_per_w

        def body(i, carry):
            off = base + i * chunk
            pltpu.sync_copy(idx_hbm.at[pl.ds(off, chunk)], idx_v)
            pltpu.async_copy(table_hbm.at[idx_v], rows_v, sem).wait()
            pltpu.sync_copy(rows_v, out_hbm.at[pl.ds(off, chunk)])
            return carry

        lax.fori_loop(0, n_chunks, body, 0)

    return emb_kernel(idx, embd)


def kernel(x, embd):
    b, f = x.shape
    _, d = embd.shape
    n_rows = b * f
    out = _emb_lookup(x.reshape(n_rows), embd, n_rows, d, 32, 1024)
    return out.reshape(b, f, d)


# speedup vs baseline: 1.3535x; 1.0127x over previous
import functools

import jax
import jax.numpy as jnp
from jax import lax
from jax.experimental import pallas as pl
from jax.experimental.pallas import tpu as pltpu
from jax.experimental.pallas import tpu_sc as plsc


def _emb_lookup(idx, embd, n_rows, d, n_workers, chunk, nbuf):
    rows_per_w = n_rows // n_workers
    n_chunks = rows_per_w // chunk
    assert rows_per_w % chunk == 0 and n_chunks >= nbuf
    mesh = plsc.VectorSubcoreMesh(core_axis_name="c", subcore_axis_name="s")

    scratch = (
        [pltpu.VMEM((chunk,), jnp.int32) for _ in range(nbuf)]
        + [pltpu.VMEM((chunk, d), jnp.float32) for _ in range(nbuf)]
        + [pltpu.SemaphoreType.DMA for _ in range(2 * nbuf)]
    )

    @functools.partial(
        pl.kernel,
        mesh=mesh,
        out_type=jax.ShapeDtypeStruct((n_rows, d), jnp.float32),
        scratch_types=scratch,
        compiler_params=pltpu.CompilerParams(use_tc_tiling_on_sc=False),
    )
    def emb_kernel(idx_hbm, table_hbm, out_hbm, *bufs):
        ib = bufs[:nbuf]
        rb = bufs[nbuf : 2 * nbuf]
        sg = bufs[2 * nbuf : 3 * nbuf]
        so = bufs[3 * nbuf :]
        wid = lax.axis_index("s") * 2 + lax.axis_index("c")
        base = wid * rows_per_w

        gathers = [None] * nbuf
        outs = [None] * nbuf

        def start_chunk(i):
            b = i % nbuf
            off = base + i * chunk
            pltpu.sync_copy(idx_hbm.at[pl.ds(off, chunk)], ib[b])
            gathers[b] = pltpu.async_copy(table_hbm.at[ib[b]], rb[b], sg[b])

        def writeback(j):
            b = j % nbuf
            off = base + j * chunk
            gathers[b].wait()
            outs[b] = pltpu.async_copy(rb[b], out_hbm.at[pl.ds(off, chunk)], so[b])

        for i in range(n_chunks):
            if i >= nbuf:
                outs[i % nbuf].wait()
            start_chunk(i)
            j = i - (nbuf - 1)
            if j >= 0:
                writeback(j)
        for j in range(n_chunks - (nbuf - 1), n_chunks):
            writeback(j)
        for b in range(nbuf):
            outs[b].wait()

    return emb_kernel(idx, embd)


def kernel(x, embd):
    b, f = x.shape
    _, d = embd.shape
    n_rows = b * f
    out = _emb_lookup(x.reshape(n_rows), embd, n_rows, d, 32, 832, 4)
    return out.reshape(b, f, d)
